# baseline (speedup 1.0000x reference)
import functools

import jax
import jax.numpy as jnp
from jax import lax
from jax.experimental import pallas as pl
from jax.experimental.pallas import tpu as pltpu
from jax.experimental.pallas import tpu_sc as plsc

LMAX = 3
CMAX = 128

_N = 50000
_D = (LMAX + 1) * CMAX
_J = CMAX * (LMAX + 1) ** 2

_NC = 2
_NS = 16
_NW = _NC * _NS

_C = 16
_NB = 3
_NCHUNK = _N // _C
_ITERS = -(-_NCHUNK // _NW)
_OUTER = -(-_ITERS // _NB)


def _sc_body(x_hbm, idx_hbm, out_hbm, idx_v, xb0, xb1, xb2, ob0, ob1, ob2,
             sin0, sin1, sin2, sout0, sout1, sout2):
    w = lax.axis_index("s") * _NC + lax.axis_index("c")
    pltpu.sync_copy(idx_hbm, idx_v)
    xbs, obs = (xb0, xb1, xb2), (ob0, ob1, ob2)
    sins, souts = (sin0, sin1, sin2), (sout0, sout1, sout2)

    def valid(k):
        return (w + k * _NW) < _NCHUNK

    def base(k):
        return (w + k * _NW) * _C

    def start_in(k, b):
        @pl.when(valid(k))
        def _():
            pltpu.async_copy(x_hbm.at[pl.ds(base(k), _C), :], xbs[b], sins[b])

    def wait_in(b):
        pltpu.make_async_copy(x_hbm.at[pl.ds(0, _C), :], xbs[b], sins[b]).wait()

    def start_out(k, b):
        pltpu.async_copy(obs[b], out_hbm.at[pl.ds(base(k), _C), :], souts[b])

    def wait_out(b):
        pltpu.make_async_copy(obs[b], out_hbm.at[pl.ds(0, _C), :], souts[b]).wait()

    def compute(b):
        xb, ob = xbs[b], obs[b]

        @plsc.parallel_loop(0, _J // 16)
        def _g(g):
            idx_vec = idx_v[pl.ds(g * 16, 16)]
            for r in range(_C):
                row = jnp.full((16,), r, jnp.int32)
                ob[r, pl.ds(g * 16, 16)] = plsc.load_gather(xb, [row, idx_vec])

    for k0 in range(_NB - 1):
        start_in(k0, k0)

    def outer(kk, carry):
        for b0 in range(_NB):
            k = kk * _NB + b0
            b = b0

            @pl.when(valid(k))
            def _():
                wait_in(b)

            @pl.when((k >= _NB) & valid(k - _NB))
            def _():
                wait_out(b)

            @pl.when(valid(k))
            def _():
                compute(b)
                start_out(k, b)

            start_in(k + _NB - 1, (b0 + _NB - 1) % _NB)

        return carry

    lax.fori_loop(0, _OUTER, outer, 0)

    for kf in range(_OUTER * _NB - _NB, _OUTER * _NB):
        @pl.when(valid(kf))
        def _():
            wait_out(kf % _NB)


def kernel(x, indices):
    n, d = x.shape
    assert n == _N and d == _D
    mesh = plsc.VectorSubcoreMesh(core_axis_name="c", subcore_axis_name="s")
    sc = functools.partial(
        pl.kernel,
        mesh=mesh,
        out_type=jax.ShapeDtypeStruct((_N, _J), jnp.float32),
        scratch_types=[
            pltpu.VMEM((_J,), jnp.int32),
            pltpu.VMEM((_C, _D), jnp.float32),
            pltpu.VMEM((_C, _D), jnp.float32),
            pltpu.VMEM((_C, _D), jnp.float32),
            pltpu.VMEM((_C, _J), jnp.float32),
            pltpu.VMEM((_C, _J), jnp.float32),
            pltpu.VMEM((_C, _J), jnp.float32),
            pltpu.SemaphoreType.DMA,
            pltpu.SemaphoreType.DMA,
            pltpu.SemaphoreType.DMA,
            pltpu.SemaphoreType.DMA,
            pltpu.SemaphoreType.DMA,
            pltpu.SemaphoreType.DMA,
        ],
        compiler_params=pltpu.CompilerParams(
            needs_layout_passes=False,
            use_tc_tiling_on_sc=True,
        ),
    )(_sc_body)
    return sc(x, indices.astype(jnp.int32))
